# baseline (speedup 1.0000x reference)
import functools

import jax
import jax.numpy as jnp
from jax.experimental import pallas as pl
from jax.experimental.pallas import tpu as pltpu

_N = 10000
_D = 128
_K = 16
_TEMP = 0.07
_EPS = 1e-8
_ROWS = 400
_CPAD = 10240
_SLOTS = 4


def _store_topk(temp, k, rows, vals, idxs, w_ref, idx_ref):
    ex = jnp.exp((vals - vals[:, 0][:, None]) / temp)
    s = jnp.sum(ex, axis=1, keepdims=True)
    w_ref[...] = ex / s * (k / (k + 1.0))
    idx_ref[...] = idxs


def _topk_softmax_body(n, k, temp, rows, cpad, slots, e_ref, et_ref, w_ref,
                       idx_ref):
    pid = pl.program_id(0)
    e = e_ref[...]
    row_g = pid * rows + jax.lax.broadcasted_iota(jnp.int32, (rows, 1), 0)
    lane128 = jax.lax.broadcasted_iota(jnp.int32, (rows, 128), 1)

    nb = cpad // 128
    neg = jnp.full((rows, 128), -jnp.inf, jnp.float32)
    rv = [neg for _ in range(slots)]
    rb = [jnp.zeros((rows, 128), jnp.int32) for _ in range(slots)]
    spill = neg
    los_v, los_b = neg, jnp.zeros((rows, 128), jnp.int32)
    for p in range(nb // 2):
        b0, b1 = 2 * p, 2 * p + 1
        va = jnp.dot(e, et_ref[:, b0 * 128:(b0 + 1) * 128],
                     preferred_element_type=jnp.float32)
        vb = jnp.dot(e, et_ref[:, b1 * 128:(b1 + 1) * 128],
                     preferred_element_type=jnp.float32)
        if (b0 + 1) * 128 > n:
            va = jnp.where(b0 * 128 + lane128 >= n, -1e9, va)
        if (b1 + 1) * 128 > n:
            vb = jnp.where(b1 * 128 + lane128 >= n, -1e9, vb)
        aw = va >= vb
        v = jnp.where(aw, va, vb)
        bb = jnp.where(aw, b0, b1).astype(jnp.int32)
        lv = jnp.where(aw, vb, va)
        lb = jnp.where(aw, b1, b0).astype(jnp.int32)
        for s in range(slots):
            keep = rv[s] >= v
            nv = jnp.where(keep, rv[s], v)
            nbk = jnp.where(keep, rb[s], bb)
            v = jnp.where(keep, v, rv[s])
            if s < slots - 1:
                bb = jnp.where(keep, bb, rb[s])
            rv[s], rb[s] = nv, nbk
        spill = jnp.maximum(spill, v)
        lkeep = los_v >= lv
        spill = jnp.maximum(spill, jnp.where(lkeep, lv, los_v))
        los_b = jnp.where(lkeep, los_b, lb)
        los_v = jnp.where(lkeep, los_v, lv)

    v, bb = los_v, los_b
    for s in range(slots):
        keep = rv[s] >= v
        nv = jnp.where(keep, rv[s], v)
        nbk = jnp.where(keep, rb[s], bb)
        v = jnp.where(keep, v, rv[s])
        bb = jnp.where(keep, bb, rb[s])
        rv[s], rb[s] = nv, nbk
    rv.append(v)
    rb.append(bb)
    depth = slots + 1
    for _ in range(depth - 1):
        for s in range(depth - 1):
            swap = (rv[s] == rv[s + 1]) & (rb[s] > rb[s + 1])
            lo = jnp.where(swap, rb[s + 1], rb[s])
            hi = jnp.where(swap, rb[s], rb[s + 1])
            rb[s], rb[s + 1] = lo, hi
    rc = [r * 128 + lane128 for r in rb]

    lane = jax.lax.broadcasted_iota(jnp.int32, (rows, k), 1)
    vals = jnp.zeros((rows, k), jnp.float32)
    idxs = jnp.zeros((rows, k), jnp.int32)
    big = jnp.full((rows, 128), cpad, jnp.int32)
    found = jnp.zeros((rows, 1), jnp.int32)
    for j in range(k + 1):
        m = jnp.max(rv[0], axis=1)
        a = jnp.min(jnp.where(rv[0] == m[:, None], rc[0], big), axis=1)
        is_self = a[:, None] == row_g
        put = (lane == j - found) & jnp.logical_not(is_self)
        vals = jnp.where(put, m[:, None], vals)
        idxs = jnp.where(put, a[:, None], idxs)
        found = found + is_self.astype(jnp.int32)
        if j < k:
            sel = (rv[0] == m[:, None]) & (rc[0] == a[:, None])
            for s in range(depth - 1):
                rv[s] = jnp.where(sel, rv[s + 1], rv[s])
                rc[s] = jnp.where(sel, rc[s + 1], rc[s])
            rv[depth - 1] = jnp.where(sel, -jnp.inf, rv[depth - 1])

    _store_topk(temp, k, rows, vals, idxs, w_ref, idx_ref)

    bad = jnp.any(spill >= vals[:, k - 1][:, None])

    @pl.when(bad)
    def _slow():
        col = jax.lax.broadcasted_iota(jnp.int32, (rows, cpad), 1)
        s2 = jnp.dot(e, et_ref[...], preferred_element_type=jnp.float32)
        s2 = jnp.where((col == row_g) | (col >= n), -1e9, s2)
        vals2 = jnp.zeros((rows, k), jnp.float32)
        idxs2 = jnp.zeros((rows, k), jnp.int32)
        m2 = jnp.max(s2, axis=1)
        for j in range(k):
            a2 = jnp.min(jnp.where(s2 == m2[:, None], col, cpad), axis=1)
            vals2 = jnp.where(lane == j, m2[:, None], vals2)
            idxs2 = jnp.where(lane == j, a2[:, None], idxs2)
            if j < k - 1:
                s3 = jnp.where(col == a2[:, None], -jnp.inf, s2)
                m2 = jnp.max(s3, axis=1)
                s2 = s3
        _store_topk(temp, k, rows, vals2, idxs2, w_ref, idx_ref)


def _graph_weights(e):
    et = jnp.zeros((_D, _CPAD), jnp.float32).at[:, :_N].set(e.T)
    grid = _N // _ROWS
    return pl.pallas_call(
        functools.partial(_topk_softmax_body, _N, _K, _TEMP, _ROWS, _CPAD,
                          _SLOTS),
        grid=(grid,),
        in_specs=[
            pl.BlockSpec((_ROWS, _D), lambda i: (i, 0)),
            pl.BlockSpec((_D, _CPAD), lambda i: (0, 0)),
        ],
        out_specs=[
            pl.BlockSpec((_ROWS, _K), lambda i: (i, 0)),
            pl.BlockSpec((_ROWS, _K), lambda i: (i, 0)),
        ],
        out_shape=[
            jax.ShapeDtypeStruct((_N, _K), jnp.float32),
            jax.ShapeDtypeStruct((_N, _K), jnp.int32),
        ],
        compiler_params=pltpu.CompilerParams(
            dimension_semantics=("arbitrary",),
        ),
    )(e, et)


def kernel(node_emb, step, detach_weights):
    nrm = jnp.sqrt(jnp.sum(node_emb * node_emb, axis=1, keepdims=True))
    e = node_emb / jnp.maximum(nrm, _EPS)
    w, knn_idx = _graph_weights(e)

    row = jnp.repeat(jnp.arange(_N, dtype=jnp.int64), _K)
    col = knn_idx.reshape(-1).astype(jnp.int64)
    val = w.reshape(-1)

    self_row = jnp.arange(_N, dtype=jnp.int64)
    self_val = jnp.full((_N,), 1.0 / (_K + 1), dtype=val.dtype)
    row = jnp.concatenate([row, self_row], axis=0)
    col = jnp.concatenate([col, self_row], axis=0)
    val = jnp.concatenate([val, self_val], axis=0)

    row0, col0, val0 = row, col, val
    row = jnp.concatenate([row0, col0], axis=0)
    col = jnp.concatenate([col0, row0], axis=0)
    val = jnp.concatenate([val0, val0], axis=0)

    indices = jnp.stack([row, col], axis=0)
    return indices, val


# speedup vs baseline: 1.0294x; 1.0294x over previous
import functools

import jax
import jax.numpy as jnp
from jax.experimental import pallas as pl
from jax.experimental.pallas import tpu as pltpu

_N = 10000
_D = 128
_K = 16
_TEMP = 0.07
_EPS = 1e-8
_ROWS = 400
_CPAD = 10240
_SLOTS = 4


def _store_topk(temp, k, rows, vals, idxs, w_ref, idx_ref):
    ex = jnp.exp((vals - vals[:, 0][:, None]) / temp)
    s = jnp.sum(ex, axis=1, keepdims=True)
    w_ref[...] = ex / s * (k / (k + 1.0))
    idx_ref[...] = idxs


def _topk_softmax_body(n, k, temp, rows, cpad, slots, e_ref, et_ref, w_ref,
                       idx_ref):
    pid = pl.program_id(0)
    e = e_ref[...]
    row_g = pid * rows + jax.lax.broadcasted_iota(jnp.int32, (rows, 1), 0)
    lane128 = jax.lax.broadcasted_iota(jnp.int32, (rows, 128), 1)

    nb = cpad // 128
    neg = jnp.full((rows, 128), -jnp.inf, jnp.float32)
    rv = [neg for _ in range(slots)]
    rb = [jnp.zeros((rows, 128), jnp.int32) for _ in range(slots)]
    spill = neg
    los_v, los_b = neg, jnp.zeros((rows, 128), jnp.int32)
    sim = jnp.dot(e, et_ref[...], preferred_element_type=jnp.float32)
    for p in range(nb // 2):
        b0, b1 = 2 * p, 2 * p + 1
        va = sim[:, b0 * 128:(b0 + 1) * 128]
        vb = sim[:, b1 * 128:(b1 + 1) * 128]
        if (b0 + 1) * 128 > n:
            va = jnp.where(b0 * 128 + lane128 >= n, -1e9, va)
        if (b1 + 1) * 128 > n:
            vb = jnp.where(b1 * 128 + lane128 >= n, -1e9, vb)
        aw = va >= vb
        v = jnp.where(aw, va, vb)
        bb = jnp.where(aw, b0, b1).astype(jnp.int32)
        lv = jnp.where(aw, vb, va)
        lb = jnp.where(aw, b1, b0).astype(jnp.int32)
        for s in range(slots):
            keep = rv[s] >= v
            nv = jnp.where(keep, rv[s], v)
            nbk = jnp.where(keep, rb[s], bb)
            v = jnp.where(keep, v, rv[s])
            if s < slots - 1:
                bb = jnp.where(keep, bb, rb[s])
            rv[s], rb[s] = nv, nbk
        spill = jnp.maximum(spill, v)
        lkeep = los_v >= lv
        spill = jnp.maximum(spill, jnp.where(lkeep, lv, los_v))
        los_b = jnp.where(lkeep, los_b, lb)
        los_v = jnp.where(lkeep, los_v, lv)

    v, bb = los_v, los_b
    for s in range(slots):
        keep = rv[s] >= v
        nv = jnp.where(keep, rv[s], v)
        nbk = jnp.where(keep, rb[s], bb)
        v = jnp.where(keep, v, rv[s])
        bb = jnp.where(keep, bb, rb[s])
        rv[s], rb[s] = nv, nbk
    rv.append(v)
    rb.append(bb)
    depth = slots + 1
    for _ in range(depth - 1):
        for s in range(depth - 1):
            swap = (rv[s] == rv[s + 1]) & (rb[s] > rb[s + 1])
            lo = jnp.where(swap, rb[s + 1], rb[s])
            hi = jnp.where(swap, rb[s], rb[s + 1])
            rb[s], rb[s + 1] = lo, hi
    rc = [r * 128 + lane128 for r in rb]

    lane = jax.lax.broadcasted_iota(jnp.int32, (rows, k), 1)
    vals = jnp.zeros((rows, k), jnp.float32)
    idxs = jnp.zeros((rows, k), jnp.int32)
    big = jnp.full((rows, 128), cpad, jnp.int32)
    found = jnp.zeros((rows, 1), jnp.int32)
    for j in range(k + 1):
        m = jnp.max(rv[0], axis=1)
        a = jnp.min(jnp.where(rv[0] == m[:, None], rc[0], big), axis=1)
        is_self = a[:, None] == row_g
        put = (lane == j - found) & jnp.logical_not(is_self)
        vals = jnp.where(put, m[:, None], vals)
        idxs = jnp.where(put, a[:, None], idxs)
        found = found + is_self.astype(jnp.int32)
        if j < k:
            sel = (rv[0] == m[:, None]) & (rc[0] == a[:, None])
            for s in range(depth - 1):
                rv[s] = jnp.where(sel, rv[s + 1], rv[s])
                rc[s] = jnp.where(sel, rc[s + 1], rc[s])
            rv[depth - 1] = jnp.where(sel, -jnp.inf, rv[depth - 1])

    _store_topk(temp, k, rows, vals, idxs, w_ref, idx_ref)

    bad = jnp.any(spill >= vals[:, k - 1][:, None])

    @pl.when(bad)
    def _slow():
        col = jax.lax.broadcasted_iota(jnp.int32, (rows, cpad), 1)
        s2 = jnp.where((col == row_g) | (col >= n), -1e9, sim)
        vals2 = jnp.zeros((rows, k), jnp.float32)
        idxs2 = jnp.zeros((rows, k), jnp.int32)
        m2 = jnp.max(s2, axis=1)
        for j in range(k):
            a2 = jnp.min(jnp.where(s2 == m2[:, None], col, cpad), axis=1)
            vals2 = jnp.where(lane == j, m2[:, None], vals2)
            idxs2 = jnp.where(lane == j, a2[:, None], idxs2)
            if j < k - 1:
                s3 = jnp.where(col == a2[:, None], -jnp.inf, s2)
                m2 = jnp.max(s3, axis=1)
                s2 = s3
        _store_topk(temp, k, rows, vals2, idxs2, w_ref, idx_ref)


def _graph_weights(e):
    et = jnp.zeros((_D, _CPAD), jnp.float32).at[:, :_N].set(e.T)
    grid = _N // _ROWS
    return pl.pallas_call(
        functools.partial(_topk_softmax_body, _N, _K, _TEMP, _ROWS, _CPAD,
                          _SLOTS),
        grid=(grid,),
        in_specs=[
            pl.BlockSpec((_ROWS, _D), lambda i: (i, 0)),
            pl.BlockSpec((_D, _CPAD), lambda i: (0, 0)),
        ],
        out_specs=[
            pl.BlockSpec((_ROWS, _K), lambda i: (i, 0)),
            pl.BlockSpec((_ROWS, _K), lambda i: (i, 0)),
        ],
        out_shape=[
            jax.ShapeDtypeStruct((_N, _K), jnp.float32),
            jax.ShapeDtypeStruct((_N, _K), jnp.int32),
        ],
        compiler_params=pltpu.CompilerParams(
            dimension_semantics=("arbitrary",),
        ),
    )(e, et)


def kernel(node_emb, step, detach_weights):
    nrm = jnp.sqrt(jnp.sum(node_emb * node_emb, axis=1, keepdims=True))
    e = node_emb / jnp.maximum(nrm, _EPS)
    w, knn_idx = _graph_weights(e)

    row = jnp.repeat(jnp.arange(_N, dtype=jnp.int64), _K)
    col = knn_idx.reshape(-1).astype(jnp.int64)
    val = w.reshape(-1)

    self_row = jnp.arange(_N, dtype=jnp.int64)
    self_val = jnp.full((_N,), 1.0 / (_K + 1), dtype=val.dtype)
    row = jnp.concatenate([row, self_row], axis=0)
    col = jnp.concatenate([col, self_row], axis=0)
    val = jnp.concatenate([val, self_val], axis=0)

    row0, col0, val0 = row, col, val
    row = jnp.concatenate([row0, col0], axis=0)
    col = jnp.concatenate([col0, row0], axis=0)
    val = jnp.concatenate([val0, val0], axis=0)

    indices = jnp.stack([row, col], axis=0)
    return indices, val


# speedup vs baseline: 1.0880x; 1.0570x over previous
import functools

import jax
import jax.numpy as jnp
from jax.experimental import pallas as pl
from jax.experimental.pallas import tpu as pltpu

_N = 10000
_D = 128
_K = 16
_TEMP = 0.07
_EPS = 1e-8
_ROWS = 256
_CPAD = 10240
_SLOTS = 4


def _store_topk(temp, k, vals, idxs, w_ref, idx_ref):
    ex = jnp.exp((vals - vals[0:1, :]) / temp)
    s = jnp.sum(ex, axis=0, keepdims=True)
    w_ref[...] = ex / s * (k / (k + 1.0))
    idx_ref[...] = idxs


def _topk_softmax_body(n, k, temp, rows, cpad, slots, ep_ref, et_ref, w_ref,
                       idx_ref):
    pid = pl.program_id(0)
    row_g = pid * rows + jax.lax.broadcasted_iota(jnp.int32, (1, rows), 1)
    sub128 = jax.lax.broadcasted_iota(jnp.int32, (128, rows), 0)
    subk = jax.lax.broadcasted_iota(jnp.int32, (k, rows), 0)

    simT = jnp.dot(ep_ref[...], et_ref[...],
                   preferred_element_type=jnp.float32)

    nb = cpad // 128
    neg = jnp.full((128, rows), -jnp.inf, jnp.float32)
    rv = [neg for _ in range(slots)]
    rb = [jnp.zeros((128, rows), jnp.int32) for _ in range(slots)]
    spill = neg
    los_v, los_b = neg, jnp.zeros((128, rows), jnp.int32)
    for p in range(nb // 2):
        b0, b1 = 2 * p, 2 * p + 1
        va = simT[b0 * 128:(b0 + 1) * 128, :]
        vb = simT[b1 * 128:(b1 + 1) * 128, :]
        if (b0 + 1) * 128 > n:
            va = jnp.where(b0 * 128 + sub128 >= n, -1e9, va)
        if (b1 + 1) * 128 > n:
            vb = jnp.where(b1 * 128 + sub128 >= n, -1e9, vb)
        aw = va >= vb
        v = jnp.where(aw, va, vb)
        bb = jnp.where(aw, b0, b1).astype(jnp.int32)
        lv = jnp.where(aw, vb, va)
        lb = jnp.where(aw, b1, b0).astype(jnp.int32)
        for s in range(slots):
            keep = rv[s] >= v
            nv = jnp.where(keep, rv[s], v)
            nbk = jnp.where(keep, rb[s], bb)
            v = jnp.where(keep, v, rv[s])
            if s < slots - 1:
                bb = jnp.where(keep, bb, rb[s])
            rv[s], rb[s] = nv, nbk
        spill = jnp.maximum(spill, v)
        lkeep = los_v >= lv
        spill = jnp.maximum(spill, jnp.where(lkeep, lv, los_v))
        los_b = jnp.where(lkeep, los_b, lb)
        los_v = jnp.where(lkeep, los_v, lv)

    v, bb = los_v, los_b
    for s in range(slots):
        keep = rv[s] >= v
        nv = jnp.where(keep, rv[s], v)
        nbk = jnp.where(keep, rb[s], bb)
        v = jnp.where(keep, v, rv[s])
        bb = jnp.where(keep, bb, rb[s])
        rv[s], rb[s] = nv, nbk
    rv.append(v)
    rb.append(bb)
    depth = slots + 1
    for _ in range(depth - 1):
        for s in range(depth - 1):
            swap = (rv[s] == rv[s + 1]) & (rb[s] > rb[s + 1])
            lo = jnp.where(swap, rb[s + 1], rb[s])
            hi = jnp.where(swap, rb[s], rb[s + 1])
            rb[s], rb[s + 1] = lo, hi
    rc = [r * 128 + sub128 for r in rb]

    vals = jnp.zeros((k, rows), jnp.float32)
    idxs = jnp.zeros((k, rows), jnp.int32)
    big = jnp.full((128, rows), cpad, jnp.int32)
    found = jnp.zeros((1, rows), jnp.int32)
    for j in range(k + 1):
        m = jnp.max(rv[0], axis=0, keepdims=True)
        a = jnp.min(jnp.where(rv[0] == m, rc[0], big), axis=0, keepdims=True)
        is_self = a == row_g
        put = (subk == j - found) & jnp.logical_not(is_self)
        vals = jnp.where(put, m, vals)
        idxs = jnp.where(put, a, idxs)
        found = found + is_self.astype(jnp.int32)
        if j < k:
            sel = (rv[0] == m) & (rc[0] == a)
            for s in range(depth - 1):
                rv[s] = jnp.where(sel, rv[s + 1], rv[s])
                rc[s] = jnp.where(sel, rc[s + 1], rc[s])
            rv[depth - 1] = jnp.where(sel, -jnp.inf, rv[depth - 1])

    _store_topk(temp, k, vals, idxs, w_ref, idx_ref)

    bad = jnp.any((spill >= vals[k - 1:k, :]) & (row_g < n))

    @pl.when(bad)
    def _slow():
        colt = jax.lax.broadcasted_iota(jnp.int32, (cpad, rows), 0)
        s2 = jnp.where((colt == row_g) | (colt >= n), -1e9, simT)
        vals2 = jnp.zeros((k, rows), jnp.float32)
        idxs2 = jnp.zeros((k, rows), jnp.int32)
        m2 = jnp.max(s2, axis=0, keepdims=True)
        for j in range(k):
            a2 = jnp.min(jnp.where(s2 == m2, colt, cpad), axis=0,
                         keepdims=True)
            vals2 = jnp.where(subk == j, m2, vals2)
            idxs2 = jnp.where(subk == j, a2, idxs2)
            if j < k - 1:
                s3 = jnp.where(colt == a2, -jnp.inf, s2)
                m2 = jnp.max(s3, axis=0, keepdims=True)
                s2 = s3
        _store_topk(temp, k, vals2, idxs2, w_ref, idx_ref)


def _graph_weights(e):
    ep = jnp.zeros((_CPAD, _D), jnp.float32).at[:_N].set(e)
    et = jnp.zeros((_D, _CPAD), jnp.float32).at[:, :_N].set(e.T)
    grid = _CPAD // _ROWS
    wt, idxt = pl.pallas_call(
        functools.partial(_topk_softmax_body, _N, _K, _TEMP, _ROWS, _CPAD,
                          _SLOTS),
        grid=(grid,),
        in_specs=[
            pl.BlockSpec((_CPAD, _D), lambda i: (0, 0)),
            pl.BlockSpec((_D, _ROWS), lambda i: (0, i)),
        ],
        out_specs=[
            pl.BlockSpec((_K, _ROWS), lambda i: (0, i)),
            pl.BlockSpec((_K, _ROWS), lambda i: (0, i)),
        ],
        out_shape=[
            jax.ShapeDtypeStruct((_K, _CPAD), jnp.float32),
            jax.ShapeDtypeStruct((_K, _CPAD), jnp.int32),
        ],
        compiler_params=pltpu.CompilerParams(
            dimension_semantics=("arbitrary",),
        ),
    )(ep, et)
    return wt.T[:_N], idxt.T[:_N]


def kernel(node_emb, step, detach_weights):
    nrm = jnp.sqrt(jnp.sum(node_emb * node_emb, axis=1, keepdims=True))
    e = node_emb / jnp.maximum(nrm, _EPS)
    w, knn_idx = _graph_weights(e)

    row = jnp.repeat(jnp.arange(_N, dtype=jnp.int64), _K)
    col = knn_idx.reshape(-1).astype(jnp.int64)
    val = w.reshape(-1)

    self_row = jnp.arange(_N, dtype=jnp.int64)
    self_val = jnp.full((_N,), 1.0 / (_K + 1), dtype=val.dtype)
    row = jnp.concatenate([row, self_row], axis=0)
    col = jnp.concatenate([col, self_row], axis=0)
    val = jnp.concatenate([val, self_val], axis=0)

    row0, col0, val0 = row, col, val
    row = jnp.concatenate([row0, col0], axis=0)
    col = jnp.concatenate([col0, row0], axis=0)
    val = jnp.concatenate([val0, val0], axis=0)

    indices = jnp.stack([row, col], axis=0)
    return indices, val


# speedup vs baseline: 1.1447x; 1.0521x over previous
import functools

import jax
import jax.numpy as jnp
from jax.experimental import pallas as pl
from jax.experimental.pallas import tpu as pltpu

_N = 10000
_D = 128
_K = 16
_TEMP = 0.07
_EPS = 1e-8
_ROWS = 256
_CPAD = 10240
_SLOTS = 4


def _store_topk(temp, k, vals, idxs, w_ref, idx_ref):
    ex = jnp.exp((vals - vals[0:1, :]) / temp)
    s = jnp.sum(ex, axis=0, keepdims=True)
    w_ref[...] = ex / s * (k / (k + 1.0))
    idx_ref[...] = idxs


def _topk_softmax_body(n, k, temp, rows, cpad, slots, ep_ref, et_ref, w_ref,
                       idx_ref):
    pid = pl.program_id(0)
    row_g = pid * rows + jax.lax.broadcasted_iota(jnp.int32, (1, rows), 1)
    sub128 = jax.lax.broadcasted_iota(jnp.int32, (128, rows), 0)
    subk = jax.lax.broadcasted_iota(jnp.int32, (k, rows), 0)

    simT = jnp.dot(ep_ref[...], et_ref[...],
                   preferred_element_type=jnp.float32)

    nb = cpad // 128
    neg = jnp.full((128, rows), -jnp.inf, jnp.float32)
    rv = [neg for _ in range(slots)]
    rb = [jnp.zeros((128, rows), jnp.int32) for _ in range(slots)]
    spill = neg
    los_v, los_b = neg, jnp.zeros((128, rows), jnp.int32)
    def blk(b):
        v = simT[b * 128:(b + 1) * 128, :]
        if (b + 1) * 128 > n:
            v = jnp.where(b * 128 + sub128 >= n, -1e9, v)
        return v

    def duel(va, ba, vb, bbk):
        aw = va >= vb
        wv = jnp.where(aw, va, vb)
        wb = jnp.where(aw, ba, bbk)
        lv = jnp.where(aw, vb, va)
        lb = jnp.where(aw, bbk, ba)
        return wv, wb, lv, lb

    def c32(b):
        return jnp.full((128, rows), b, jnp.int32)

    for q in range(nb // 4):
        b0 = 4 * q
        w01, w01b, l01, l01b = duel(blk(b0), c32(b0), blk(b0 + 1), c32(b0 + 1))
        w23, w23b, l23, l23b = duel(blk(b0 + 2), c32(b0 + 2), blk(b0 + 3),
                                    c32(b0 + 3))
        v, bb, lf, lfb = duel(w01, w01b, w23, w23b)
        for s in range(slots):
            keep = rv[s] >= v
            nv = jnp.where(keep, rv[s], v)
            nbk = jnp.where(keep, rb[s], bb)
            v = jnp.where(keep, v, rv[s])
            if s < slots - 1:
                bb = jnp.where(keep, bb, rb[s])
            rv[s], rb[s] = nv, nbk
        spill = jnp.maximum(spill, v)
        for lv, lb in ((l01, l01b), (l23, l23b), (lf, lfb)):
            lkeep = los_v >= lv
            spill = jnp.maximum(spill, jnp.where(lkeep, lv, los_v))
            los_b = jnp.where(lkeep, los_b, lb)
            los_v = jnp.where(lkeep, los_v, lv)

    v, bb = los_v, los_b
    for s in range(slots):
        keep = rv[s] >= v
        nv = jnp.where(keep, rv[s], v)
        nbk = jnp.where(keep, rb[s], bb)
        v = jnp.where(keep, v, rv[s])
        bb = jnp.where(keep, bb, rb[s])
        rv[s], rb[s] = nv, nbk
    rv.append(v)
    rb.append(bb)
    depth = slots + 1
    for _ in range(depth - 1):
        for s in range(depth - 1):
            swap = (rv[s] == rv[s + 1]) & (rb[s] > rb[s + 1])
            lo = jnp.where(swap, rb[s + 1], rb[s])
            hi = jnp.where(swap, rb[s], rb[s + 1])
            rb[s], rb[s + 1] = lo, hi
    rc = [r * 128 + sub128 for r in rb]

    vals = jnp.zeros((k, rows), jnp.float32)
    idxs = jnp.zeros((k, rows), jnp.int32)
    big = jnp.full((128, rows), cpad, jnp.int32)
    found = jnp.zeros((1, rows), jnp.int32)
    for j in range(k + 1):
        m = jnp.max(rv[0], axis=0, keepdims=True)
        a = jnp.min(jnp.where(rv[0] == m, rc[0], big), axis=0, keepdims=True)
        is_self = a == row_g
        put = (subk == j - found) & jnp.logical_not(is_self)
        vals = jnp.where(put, m, vals)
        idxs = jnp.where(put, a, idxs)
        found = found + is_self.astype(jnp.int32)
        if j < k:
            sel = (rv[0] == m) & (rc[0] == a)
            for s in range(depth - 1):
                rv[s] = jnp.where(sel, rv[s + 1], rv[s])
                rc[s] = jnp.where(sel, rc[s + 1], rc[s])
            rv[depth - 1] = jnp.where(sel, -jnp.inf, rv[depth - 1])

    _store_topk(temp, k, vals, idxs, w_ref, idx_ref)

    bad = jnp.any((spill >= vals[k - 1:k, :]) & (row_g < n))

    @pl.when(bad)
    def _slow():
        colt = jax.lax.broadcasted_iota(jnp.int32, (cpad, rows), 0)
        s2 = jnp.where((colt == row_g) | (colt >= n), -1e9, simT)
        vals2 = jnp.zeros((k, rows), jnp.float32)
        idxs2 = jnp.zeros((k, rows), jnp.int32)
        m2 = jnp.max(s2, axis=0, keepdims=True)
        for j in range(k):
            a2 = jnp.min(jnp.where(s2 == m2, colt, cpad), axis=0,
                         keepdims=True)
            vals2 = jnp.where(subk == j, m2, vals2)
            idxs2 = jnp.where(subk == j, a2, idxs2)
            if j < k - 1:
                s3 = jnp.where(colt == a2, -jnp.inf, s2)
                m2 = jnp.max(s3, axis=0, keepdims=True)
                s2 = s3
        _store_topk(temp, k, vals2, idxs2, w_ref, idx_ref)


def _graph_weights(e):
    ep = jnp.zeros((_CPAD, _D), jnp.float32).at[:_N].set(e)
    et = jnp.zeros((_D, _CPAD), jnp.float32).at[:, :_N].set(e.T)
    grid = _CPAD // _ROWS
    wt, idxt = pl.pallas_call(
        functools.partial(_topk_softmax_body, _N, _K, _TEMP, _ROWS, _CPAD,
                          _SLOTS),
        grid=(grid,),
        in_specs=[
            pl.BlockSpec((_CPAD, _D), lambda i: (0, 0)),
            pl.BlockSpec((_D, _ROWS), lambda i: (0, i)),
        ],
        out_specs=[
            pl.BlockSpec((_K, _ROWS), lambda i: (0, i)),
            pl.BlockSpec((_K, _ROWS), lambda i: (0, i)),
        ],
        out_shape=[
            jax.ShapeDtypeStruct((_K, _CPAD), jnp.float32),
            jax.ShapeDtypeStruct((_K, _CPAD), jnp.int32),
        ],
        compiler_params=pltpu.CompilerParams(
            dimension_semantics=("arbitrary",),
        ),
    )(ep, et)
    return wt.T[:_N], idxt.T[:_N]


def kernel(node_emb, step, detach_weights):
    nrm = jnp.sqrt(jnp.sum(node_emb * node_emb, axis=1, keepdims=True))
    e = node_emb / jnp.maximum(nrm, _EPS)
    w, knn_idx = _graph_weights(e)

    row = jnp.repeat(jnp.arange(_N, dtype=jnp.int64), _K)
    col = knn_idx.reshape(-1).astype(jnp.int64)
    val = w.reshape(-1)

    self_row = jnp.arange(_N, dtype=jnp.int64)
    self_val = jnp.full((_N,), 1.0 / (_K + 1), dtype=val.dtype)
    row = jnp.concatenate([row, self_row], axis=0)
    col = jnp.concatenate([col, self_row], axis=0)
    val = jnp.concatenate([val, self_val], axis=0)

    row0, col0, val0 = row, col, val
    row = jnp.concatenate([row0, col0], axis=0)
    col = jnp.concatenate([col0, row0], axis=0)
    val = jnp.concatenate([val0, val0], axis=0)

    indices = jnp.stack([row, col], axis=0)
    return indices, val
